# baseline (speedup 1.0000x reference)
import functools

import jax
import jax.numpy as jnp
from jax import lax
from jax.experimental import pallas as pl
from jax.experimental.pallas import tpu as pltpu
from jax.experimental.pallas import tpu_sc as plsc

SEQ = 8192
EMB = 768
L = 16
NVEC = EMB // L
NC = 2
NS = 16
NW = NC * NS
TOK_PER_W = SEQ // NW
CH = 32
NCH = TOK_PER_W // CH
RG = CH // L
CU = 4
NG = 12
NNG = NVEC // NG
EPS = 1e-5


def _rsqrt(v):
    i = lax.bitcast_convert_type(v, jnp.int32)
    i = 0x5F3759DF - lax.shift_right_arithmetic(i, 1)
    y = lax.bitcast_convert_type(i, jnp.float32)
    for _ in range(3):
        y = y * (1.5 - 0.5 * v * y * y)
    return y


_mesh = plsc.VectorSubcoreMesh(core_axis_name="c", subcore_axis_name="s")


@functools.partial(
    pl.kernel,
    mesh=_mesh,
    compiler_params=pltpu.CompilerParams(needs_layout_passes=False),
    out_type=jax.ShapeDtypeStruct((SEQ, EMB), jnp.float32),
    scratch_types=[
        pltpu.VMEM((TOK_PER_W,), jnp.int32),
        pltpu.VMEM((CH, EMB), jnp.float32),
        pltpu.VMEM((CH, EMB), jnp.float32),
        pltpu.VMEM((CH, EMB), jnp.float32),
        pltpu.VMEM((EMB,), jnp.float32),
        pltpu.VMEM((EMB,), jnp.float32),
        pltpu.VMEM((CH,), jnp.float32),
        pltpu.VMEM((CH,), jnp.float32),
        pltpu.SemaphoreType.DMA,
        pltpu.SemaphoreType.DMA,
        pltpu.SemaphoreType.DMA,
        pltpu.SemaphoreType.DMA,
        pltpu.SemaphoreType.DMA,
    ],
)
def _encode(ids_hbm, tab_hbm, pos_hbm, gam_hbm, bet_hbm, out_hbm,
            idx_v, tok0, tok1, pos_v, gam_v, bet_v, sc_v, of_v,
            ts0, ts1, ps0, os0, os1):
    wid = lax.axis_index("s") * NC + lax.axis_index("c")
    base = wid * TOK_PER_W
    tok = (tok0, tok1)
    tsem = (ts0, ts1)
    osem = (os0, os1)

    pltpu.sync_copy(ids_hbm.at[pl.ds(base, TOK_PER_W)], idx_v)
    pltpu.sync_copy(gam_hbm, gam_v)
    pltpu.sync_copy(bet_hbm, bet_v)

    def start_tok(c):
        return pltpu.async_copy(
            tab_hbm.at[idx_v.at[pl.ds(c * CH, CH)]], tok[c % 2], tsem[c % 2])

    def start_pos(c):
        return pltpu.async_copy(
            pos_hbm.at[pl.ds(base + c * CH, CH)], pos_v, ps0)

    h_tok = [None, None]
    h_out = [None, None]
    h_tok[0] = start_tok(0)
    h_pos = start_pos(0)

    for c in range(NCH):
        b = c % 2
        if c + 1 < NCH:
            nb = 1 - b
            if h_out[nb] is not None:
                h_out[nb].wait()
                h_out[nb] = None
            h_tok[nb] = start_tok(c + 1)
        h_tok[b].wait()
        h_pos.wait()

        x_v = tok[b]

        for g2 in range(RG):
            row_idx = lax.iota(jnp.int32, L) + (g2 * L)
            z = jnp.zeros((L,), jnp.float32)

            @plsc.parallel_loop(0, EMB, step=CU,
                                carry=(z, z, z, z, z, z, z, z))
            def _cols(col, acc):
                acc = list(acc)
                for u in range(CU):
                    ci = jnp.full((L,), col + u, jnp.int32)
                    t = plsc.load_gather(x_v, [row_idx, ci])
                    p = plsc.load_gather(pos_v, [row_idx, ci])
                    x = t + p
                    plsc.store_scatter(x_v, [row_idx, ci], x)
                    acc[u] = acc[u] + x
                    acc[CU + u] = acc[CU + u] + x * x
                return tuple(acc)

            accs = _cols
            s = (accs[0] + accs[1]) + (accs[2] + accs[3])
            ss = (accs[4] + accs[5]) + (accs[6] + accs[7])
            mean = s * (1.0 / EMB)
            ex2 = ss * (1.0 / EMB)
            inv = _rsqrt(ex2 - mean * mean + EPS)
            sc_v[pl.ds(g2 * L, L)] = inv
            of_v[pl.ds(g2 * L, L)] = -mean * inv

        if c + 1 < NCH:
            h_pos = start_pos(c + 1)

        for g in range(NNG):
            gbase = g * (NG * L)
            gams = [gam_v[pl.ds(gbase + jj * L, L)] for jj in range(NG)]
            bets = [bet_v[pl.ds(gbase + jj * L, L)] for jj in range(NG)]

            @plsc.parallel_loop(0, CH)
            def _norm(r):
                rg = (r // L) * L
                lane = jnp.full((L,), r - rg, jnp.int32)
                inv = sc_v[pl.ds(rg, L)].at[lane].get(mode="promise_in_bounds")
                c2 = of_v[pl.ds(rg, L)].at[lane].get(mode="promise_in_bounds")
                for jj in range(NG):
                    sl = pl.ds(gbase + jj * L, L)
                    x_v[r, sl] = (x_v[r, sl] * inv + c2) * gams[jj] + bets[jj]

        h_out[b] = pltpu.async_copy(
            x_v, out_hbm.at[pl.ds(base + c * CH, CH)], osem[b])

    for h in h_out:
        if h is not None:
            h.wait()


def kernel(token_ids, token_table, pos_table, ln_gamma, ln_beta):
    out = _encode(token_ids.astype(jnp.int32), token_table, pos_table,
                  ln_gamma, ln_beta)
    return out[None]


# speedup vs baseline: 2.2449x; 2.2449x over previous
import functools

import jax
import jax.numpy as jnp
from jax import lax
from jax.experimental import pallas as pl
from jax.experimental.pallas import tpu as pltpu
from jax.experimental.pallas import tpu_sc as plsc

SEQ = 8192
EMB = 768
L = 16
NVEC = EMB // L
NC = 2
NS = 16
NW = NC * NS
TOK_PER_W = SEQ // NW
CH = 32
NCH = TOK_PER_W // CH
JG = 12
NJG = NVEC // JG
FB = 4
EPS = 1e-5


def _lane_sum(v):
    lanes = lax.iota(jnp.int32, L)
    for k in (8, 4, 2, 1):
        v = v + v.at[lanes ^ k].get(mode="promise_in_bounds")
    return v


def _rsqrt(v):
    i = lax.bitcast_convert_type(v, jnp.int32)
    i = 0x5F3759DF - lax.shift_right_arithmetic(i, 1)
    y = lax.bitcast_convert_type(i, jnp.float32)
    for _ in range(3):
        y = y * (1.5 - 0.5 * v * y * y)
    return y


_mesh = plsc.VectorSubcoreMesh(core_axis_name="c", subcore_axis_name="s")


@functools.partial(
    pl.kernel,
    mesh=_mesh,
    out_type=jax.ShapeDtypeStruct((SEQ, EMB), jnp.float32),
    scratch_types=[
        pltpu.VMEM((TOK_PER_W,), jnp.int32),
        pltpu.VMEM((CH, EMB), jnp.float32),
        pltpu.VMEM((CH, EMB), jnp.float32),
        pltpu.VMEM((CH, EMB), jnp.float32),
        pltpu.VMEM((CH, EMB), jnp.float32),
        pltpu.VMEM((EMB,), jnp.float32),
        pltpu.VMEM((EMB,), jnp.float32),
        pltpu.VMEM((CH, L), jnp.float32),
        pltpu.VMEM((CH, L), jnp.float32),
        pltpu.VMEM((CH, L), jnp.float32),
        pltpu.VMEM((CH, L), jnp.float32),
        pltpu.SemaphoreType.DMA,
        pltpu.SemaphoreType.DMA,
        pltpu.SemaphoreType.DMA,
        pltpu.SemaphoreType.DMA,
        pltpu.SemaphoreType.DMA,
    ],
)
def _encode(ids_hbm, tab_hbm, pos_hbm, gam_hbm, bet_hbm, out_hbm,
            idx_v, tok0, tok1, pos_v, x_b, gam_v, bet_v,
            sb_v, qb_v, sc_v, of_v,
            ts0, ts1, ps0, os0, os1):
    wid = lax.axis_index("s") * NC + lax.axis_index("c")
    base = wid * TOK_PER_W
    tok = (tok0, tok1)
    tsem = (ts0, ts1)
    osem = (os0, os1)

    pltpu.sync_copy(ids_hbm.at[pl.ds(base, TOK_PER_W)], idx_v)
    pltpu.sync_copy(gam_hbm, gam_v)
    pltpu.sync_copy(bet_hbm, bet_v)

    def start_tok(c):
        return pltpu.async_copy(
            tab_hbm.at[idx_v.at[pl.ds(c * CH, CH)]], tok[c % 2], tsem[c % 2])

    def start_pos(c):
        return pltpu.async_copy(
            pos_hbm.at[pl.ds(base + c * CH, CH)], pos_v, ps0)

    h_tok = [None, None]
    h_out = [None, None]
    h_tok[0] = start_tok(0)
    h_pos = start_pos(0)

    for c in range(NCH):
        b = c % 2
        if c + 1 < NCH:
            nb = 1 - b
            if h_out[nb] is not None:
                h_out[nb].wait()
                h_out[nb] = None
            h_tok[nb] = start_tok(c + 1)
        h_tok[b].wait()
        h_pos.wait()

        t_v = tok[b]

        def _stats(r, carry):
            def _grp(g, acc):
                s0, s1, q0, q1 = acc
                gb = g * (JG * L)
                for jj in range(JG):
                    sl = pl.ds(gb + jj * L, L)
                    x = t_v[r, sl] + pos_v[r, sl]
                    x_b[r, sl] = x
                    if jj & 1:
                        s1 = s1 + x
                        q1 = q1 + x * x
                    else:
                        s0 = s0 + x
                        q0 = q0 + x * x
                return s0, s1, q0, q1

            z = jnp.zeros((L,), jnp.float32)
            s0, s1, q0, q1 = lax.fori_loop(0, NJG, _grp, (z, z, z, z))
            sb_v[r] = s0 + s1
            qb_v[r] = q0 + q1
            return carry

        lax.fori_loop(0, CH, _stats, 0)

        def _fin(q, carry):
            for u in range(FB):
                r = q * FB + u
                mean = _lane_sum(sb_v[r]) * (1.0 / EMB)
                ex2 = _lane_sum(qb_v[r]) * (1.0 / EMB)
                inv = _rsqrt(ex2 - mean * mean + EPS)
                sc_v[r] = inv
                of_v[r] = -mean * inv
            return carry

        lax.fori_loop(0, CH // FB, _fin, 0)

        if c + 1 < NCH:
            h_pos = start_pos(c + 1)

        for g in range(NJG):
            gb = g * (JG * L)
            gams = [gam_v[pl.ds(gb + jj * L, L)] for jj in range(JG)]
            bets = [bet_v[pl.ds(gb + jj * L, L)] for jj in range(JG)]

            def _norm(r, carry):
                inv = sc_v[r]
                c2 = of_v[r]
                for jj in range(JG):
                    sl = pl.ds(gb + jj * L, L)
                    t_v[r, sl] = (x_b[r, sl] * inv + c2) * gams[jj] + bets[jj]
                return carry

            lax.fori_loop(0, CH, _norm, 0)

        h_out[b] = pltpu.async_copy(
            t_v, out_hbm.at[pl.ds(base + c * CH, CH)], osem[b])

    for h in h_out:
        if h is not None:
            h.wait()


def kernel(token_ids, token_table, pos_table, ln_gamma, ln_beta):
    out = _encode(token_ids.astype(jnp.int32), token_table, pos_table,
                  ln_gamma, ln_beta)
    return out[None]


# speedup vs baseline: 3.1271x; 1.3930x over previous
import functools

import jax
import jax.numpy as jnp
from jax import lax
from jax.experimental import pallas as pl
from jax.experimental.pallas import tpu as pltpu
from jax.experimental.pallas import tpu_sc as plsc

SEQ = 8192
EMB = 768
L = 16
NVEC = EMB // L
NC = 2
NS = 16
NW = NC * NS
TOK_PER_W = SEQ // NW
CH = 32
NCH = TOK_PER_W // CH
JG = 12
NJG = NVEC // JG
NG = 12
NNG = NVEC // NG
EPS = 1e-5


def _lane_sum(v):
    lanes = lax.iota(jnp.int32, L)
    for k in (8, 4, 2, 1):
        v = v + v.at[lanes ^ k].get(mode="promise_in_bounds")
    return v


def _rsqrt(v):
    i = lax.bitcast_convert_type(v, jnp.int32)
    i = 0x5F3759DF - lax.shift_right_arithmetic(i, 1)
    y = lax.bitcast_convert_type(i, jnp.float32)
    for _ in range(3):
        y = y * (1.5 - 0.5 * v * y * y)
    return y


_mesh = plsc.VectorSubcoreMesh(core_axis_name="c", subcore_axis_name="s")


@functools.partial(
    pl.kernel,
    mesh=_mesh,
    out_type=jax.ShapeDtypeStruct((SEQ, EMB), jnp.float32),
    scratch_types=[
        pltpu.VMEM((TOK_PER_W,), jnp.int32),
        pltpu.VMEM((CH, EMB), jnp.float32),
        pltpu.VMEM((CH, EMB), jnp.float32),
        pltpu.VMEM((CH, EMB), jnp.float32),
        pltpu.VMEM((EMB,), jnp.float32),
        pltpu.VMEM((EMB,), jnp.float32),
        pltpu.VMEM((CH, L), jnp.float32),
        pltpu.VMEM((CH, L), jnp.float32),
        pltpu.SemaphoreType.DMA,
        pltpu.SemaphoreType.DMA,
        pltpu.SemaphoreType.DMA,
        pltpu.SemaphoreType.DMA,
        pltpu.SemaphoreType.DMA,
    ],
)
def _encode(ids_hbm, tab_hbm, pos_hbm, gam_hbm, bet_hbm, out_hbm,
            idx_v, tok0, tok1, pos_v, gam_v, bet_v, sc_v, of_v,
            ts0, ts1, ps0, os0, os1):
    wid = lax.axis_index("s") * NC + lax.axis_index("c")
    base = wid * TOK_PER_W
    tok = (tok0, tok1)
    tsem = (ts0, ts1)
    osem = (os0, os1)

    pltpu.sync_copy(ids_hbm.at[pl.ds(base, TOK_PER_W)], idx_v)
    pltpu.sync_copy(gam_hbm, gam_v)
    pltpu.sync_copy(bet_hbm, bet_v)

    def start_tok(c):
        return pltpu.async_copy(
            tab_hbm.at[idx_v.at[pl.ds(c * CH, CH)]], tok[c % 2], tsem[c % 2])

    def start_pos(c):
        return pltpu.async_copy(
            pos_hbm.at[pl.ds(base + c * CH, CH)], pos_v, ps0)

    h_tok = [None, None]
    h_out = [None, None]
    h_tok[0] = start_tok(0)
    h_pos = start_pos(0)

    for c in range(NCH):
        b = c % 2
        if c + 1 < NCH:
            nb = 1 - b
            if h_out[nb] is not None:
                h_out[nb].wait()
                h_out[nb] = None
            h_tok[nb] = start_tok(c + 1)
        h_tok[b].wait()
        h_pos.wait()

        x_v = tok[b]

        @plsc.parallel_loop(0, CH)
        def _stats(r):
            def _grp(g, acc):
                s, ss = acc
                gbase = g * (JG * L)
                for jj in range(JG):
                    sl = pl.ds(gbase + jj * L, L)
                    x = x_v[r, sl] + pos_v[r, sl]
                    s = s + x
                    ss = ss + x * x
                return s, ss

            z = jnp.zeros((L,), jnp.float32)
            s, ss = lax.fori_loop(0, NJG, _grp, (z, z))
            mean = _lane_sum(s) * (1.0 / EMB)
            ex2 = _lane_sum(ss) * (1.0 / EMB)
            inv = _rsqrt(ex2 - mean * mean + EPS)
            sc_v[r] = inv
            of_v[r] = -mean * inv

        for g in range(NNG):
            gbase = g * (NG * L)
            gams = [gam_v[pl.ds(gbase + jj * L, L)] for jj in range(NG)]
            bets = [bet_v[pl.ds(gbase + jj * L, L)] for jj in range(NG)]

            @plsc.parallel_loop(0, CH)
            def _norm(r):
                inv = sc_v[r]
                c2 = of_v[r]
                for jj in range(NG):
                    sl = pl.ds(gbase + jj * L, L)
                    x = x_v[r, sl] + pos_v[r, sl]
                    x_v[r, sl] = (x * inv + c2) * gams[jj] + bets[jj]

        if c + 1 < NCH:
            h_pos = start_pos(c + 1)

        h_out[b] = pltpu.async_copy(
            x_v, out_hbm.at[pl.ds(base + c * CH, CH)], osem[b])

    for h in h_out:
        if h is not None:
            h.wait()


def kernel(token_ids, token_table, pos_table, ln_gamma, ln_beta):
    out = _encode(token_ids.astype(jnp.int32), token_table, pos_table,
                  ln_gamma, ln_beta)
    return out[None]
